# baseline (speedup 1.0000x reference)
import functools

import jax
import jax.numpy as jnp
from jax import lax
from jax.experimental import pallas as pl
from jax.experimental.pallas import tpu as pltpu
from jax.experimental.pallas import tpu_sc as plsc

ROWS = 16384
COLS = 1000
LANES = 16

_INFO = plsc.get_sparse_core_info()
_NC = _INFO.num_cores
_NS = _INFO.num_subcores
_NW = _NC * _NS
ROWS_PER_W = ROWS // _NW
IDX_CHUNK = 128
N_CHUNKS = ROWS_PER_W // IDX_CHUNK


@functools.partial(
    pl.kernel,
    out_type=jax.ShapeDtypeStruct((ROWS,), jnp.float32),
    mesh=plsc.VectorSubcoreMesh(core_axis_name="c", subcore_axis_name="s"),
    compiler_params=pltpu.CompilerParams(use_tc_tiling_on_sc=False),
    scratch_types=[
        pltpu.VMEM((ROWS_PER_W,), jnp.int32),
        pltpu.VMEM((ROWS_PER_W,), jnp.float32),
        pltpu.SemaphoreType.DMA,
    ],
)
def _depth_gather(x_hbm, lab_hbm, out_hbm, idx_v, val_v, sem):
    wid = lax.axis_index("s") * _NC + lax.axis_index("c")
    base = wid * ROWS_PER_W

    pltpu.sync_copy(lab_hbm.at[pl.ds(base, ROWS_PER_W)], idx_v)

    lane = lax.iota(jnp.int32, LANES)

    def build_body(j, carry):
        labs = idx_v[pl.ds(j * LANES, LANES)]
        row = base + j * LANES + lane
        idx_v[pl.ds(j * LANES, LANES)] = row * COLS + labs
        return carry

    lax.fori_loop(0, ROWS_PER_W // LANES, build_body, 0)

    copies = [
        pltpu.async_copy(
            x_hbm.at[idx_v.at[pl.ds(r * IDX_CHUNK, IDX_CHUNK)]],
            val_v.at[pl.ds(r * IDX_CHUNK, IDX_CHUNK)],
            sem,
        )
        for r in range(N_CHUNKS)
    ]
    for cp in copies:
        cp.wait()

    def post_body(j, carry):
        v = val_v[pl.ds(j * LANES, LANES)]
        val_v[pl.ds(j * LANES, LANES)] = jnp.exp(jnp.abs(v) * 0.1) - 1.0
        return carry

    lax.fori_loop(0, ROWS_PER_W // LANES, post_body, 0)

    pltpu.sync_copy(val_v, out_hbm.at[pl.ds(base, ROWS_PER_W)])


def kernel(x, labels):
    out = _depth_gather(x.reshape(-1), labels.astype(jnp.int32))
    return out[:, None]


# speedup vs baseline: 1.3677x; 1.3677x over previous
import functools

import jax
import jax.numpy as jnp
from jax import lax
from jax.experimental import pallas as pl
from jax.experimental.pallas import tpu as pltpu
from jax.experimental.pallas import tpu_sc as plsc

ROWS = 16384
COLS = 1000
LANES = 16
BLK_W = 128
N_BLK = 8
WIN = 32
PAD_ROWS = 768

_INFO = plsc.get_sparse_core_info()
_NC = _INFO.num_cores
_NS = _INFO.num_subcores
_NW = _NC * _NS
ROWS_PER_W = ROWS // _NW
N_CHUNK = ROWS_PER_W // LANES


@functools.partial(
    pl.kernel,
    out_type=jax.ShapeDtypeStruct((ROWS,), jnp.float32),
    mesh=plsc.VectorSubcoreMesh(core_axis_name="c", subcore_axis_name="s"),
    compiler_params=pltpu.CompilerParams(needs_layout_passes=False),
    scratch_types=[
        pltpu.VMEM((ROWS_PER_W,), jnp.int32),
        pltpu.VMEM((PAD_ROWS,), jnp.int32),
        pltpu.VMEM((ROWS_PER_W,), jnp.int32),
        pltpu.VMEM((PAD_ROWS, BLK_W), jnp.float32),
        pltpu.VMEM((ROWS_PER_W,), jnp.float32),
        pltpu.SemaphoreType.DMA,
    ],
)
def _depth_gather(
    x_hbm, lab_hbm, out_hbm, lab_v, perm_v, pos_v, val_v, res_v, sem
):
    wid = lax.axis_index("s") * _NC + lax.axis_index("c")
    base = wid * ROWS_PER_W
    lane = lax.iota(jnp.int32, LANES)

    pltpu.sync_copy(lab_hbm.at[pl.ds(base, ROWS_PER_W)], lab_v)

    def fill_body(j, carry):
        perm_v[pl.ds(j * LANES, LANES)] = jnp.zeros((LANES,), jnp.int32)
        return carry

    lax.fori_loop(0, PAD_ROWS // LANES, fill_body, 0)

    def count_body(j, cnts):
        blk = lab_v[pl.ds(j * LANES, LANES)] >> 7
        return tuple(
            cnts[b] + lax.reduce_sum((blk == b).astype(jnp.int32), axes=(0,))
            for b in range(N_BLK)
        )

    zero = jnp.int32(0)
    cnts = lax.fori_loop(0, N_CHUNK, count_body, (zero,) * N_BLK)

    starts = []
    acc = zero
    for b in range(N_BLK):
        starts.append(acc)
        acc = acc + ((cnts[b] + (WIN - 1)) & (-WIN))

    def place_body(j, offs):
        labs = lab_v[pl.ds(j * LANES, LANES)]
        blk = labs >> 7
        grow = base + j * LANES + lane
        pos_acc = jnp.zeros((LANES,), jnp.int32)
        new_offs = []
        for b in range(N_BLK):
            mask = blk == b
            mi = mask.astype(jnp.int32)
            rank = plsc.cumsum(mi) - 1
            pos = offs[b] + rank
            pos_acc = jnp.where(mask, pos, pos_acc)
            plsc.store_scatter(perm_v, [pos], grow, mask=mask)
            new_offs.append(offs[b] + lax.reduce_sum(mi, axes=(0,)))
        pos_v[pl.ds(j * LANES, LANES)] = pos_acc
        return tuple(new_offs)

    lax.fori_loop(0, N_CHUNK, place_body, tuple(starts))

    def win_refs(b, k, start_b):
        w0 = pl.multiple_of(start_b + k * WIN, WIN)
        src = x_hbm.at[
            plsc.Indices(perm_v.at[pl.ds(w0, WIN)], ignored_value=-1),
            pl.ds(pl.multiple_of(b * BLK_W + 0 * w0, BLK_W), BLK_W),
        ]
        dst = val_v.at[pl.ds(w0, WIN), :]
        return src, dst

    for b in range(N_BLK):
        padlen_b = (cnts[b] + (WIN - 1)) & (-WIN)

        def issue_body(k, carry, _b=b, _s=starts[b], _n=padlen_b):
            @pl.when(k * WIN < _n)
            def _():
                src, dst = win_refs(_b, k, _s)
                pltpu.async_copy(src, dst, sem)

            return carry

        lax.fori_loop(0, ROWS_PER_W // WIN, issue_body, 0)

    for b in range(N_BLK):
        padlen_b = (cnts[b] + (WIN - 1)) & (-WIN)

        def drain_body(k, carry, _b=b, _s=starts[b], _n=padlen_b):
            @pl.when(k * WIN < _n)
            def _():
                src, dst = win_refs(_b, k, _s)
                pltpu.make_async_copy(src, dst, sem).wait()

            return carry

        lax.fori_loop(0, ROWS_PER_W // WIN, drain_body, 0)

    def extract_body(j, carry):
        labs = lab_v[pl.ds(j * LANES, LANES)]
        pos = pos_v[pl.ds(j * LANES, LANES)]
        v = plsc.load_gather(val_v, [pos, labs & (BLK_W - 1)])
        res_v[pl.ds(j * LANES, LANES)] = jnp.exp(jnp.abs(v) * 0.1) - 1.0
        return carry

    lax.fori_loop(0, N_CHUNK, extract_body, 0)

    pltpu.sync_copy(res_v, out_hbm.at[pl.ds(base, ROWS_PER_W)])


def kernel(x, labels):
    out = _depth_gather(x, labels.astype(jnp.int32))
    return out[:, None]
